# Initial kernel scaffold; baseline (speedup 1.0000x reference)
import jax
import jax.numpy as jnp
from jax.experimental import pallas as pl


def kernel(x, emb_table, W, b):
    raise NotImplementedError("write your pallas kernel here")


# speedup vs baseline: 2.2903x; 2.2903x over previous
import functools

import jax
import jax.numpy as jnp
from jax import lax
from jax.experimental import pallas as pl
from jax.experimental.pallas import tpu as pltpu
from jax.experimental.pallas import tpu_sc as plsc

B = 16384
L = 50
D = 64
V = 1000
VPAD = 1024

_info = plsc.get_sparse_core_info()
NC = _info.num_cores
NS = _info.num_subcores
NW = NC * NS
BPW = B // NW
CB = 16
NCHUNK = BPW // CB
IDX_PER_CHUNK = CB * L

_mesh = plsc.VectorSubcoreMesh(core_axis_name="c", subcore_axis_name="s")


@functools.partial(
    pl.kernel,
    mesh=_mesh,
    out_type=jax.ShapeDtypeStruct((B, D), jnp.float32),
    scratch_types=[
        pltpu.VMEM((IDX_PER_CHUNK,), jnp.int32),
        pltpu.VMEM((IDX_PER_CHUNK, D), jnp.float32),
        pltpu.VMEM((CB, D), jnp.float32),
        pltpu.SemaphoreType.DMA,
    ],
    compiler_params=pltpu.CompilerParams(use_tc_tiling_on_sc=False),
)
def _sc_gather_mean(idx_hbm, table_hbm, out_hbm, idx_v, rows_v, acc_v, sem):
    wid = lax.axis_index("s") * NC + lax.axis_index("c")

    def chunk(c, carry):
        row0 = wid * BPW + c * CB
        pltpu.sync_copy(idx_hbm.at[pl.ds(row0 * L, IDX_PER_CHUNK)], idx_v)
        pltpu.async_copy(table_hbm.at[idx_v], rows_v, sem).wait()

        def rowloop(i, carry2):
            def jloop(j, accs):
                base = i * L + j
                return tuple(
                    accs[t] + rows_v[base, pl.ds(t * 16, 16)] for t in range(4)
                )

            accs = lax.fori_loop(
                0, L, jloop, tuple(jnp.zeros((16,), jnp.float32) for _ in range(4))
            )
            for t in range(4):
                acc_v[i, pl.ds(t * 16, 16)] = accs[t] * (1.0 / L)
            return carry2

        lax.fori_loop(0, CB, rowloop, 0)
        pltpu.sync_copy(acc_v, out_hbm.at[pl.ds(row0, CB)])
        return carry

    lax.fori_loop(0, NCHUNK, chunk, 0)


def _mm_body(avg_ref, wt_ref, b_ref, o_ref):
    o_ref[...] = (
        jnp.dot(avg_ref[...], wt_ref[...], preferred_element_type=jnp.float32)
        + b_ref[...]
    )


_BM = 2048


def _tc_matmul(avg, wt, b2):
    return pl.pallas_call(
        _mm_body,
        grid=(B // _BM,),
        in_specs=[
            pl.BlockSpec((_BM, D), lambda i: (i, 0)),
            pl.BlockSpec((D, VPAD), lambda i: (0, 0)),
            pl.BlockSpec((1, VPAD), lambda i: (0, 0)),
        ],
        out_specs=pl.BlockSpec((_BM, VPAD), lambda i: (i, 0)),
        out_shape=jax.ShapeDtypeStruct((B, VPAD), jnp.float32),
    )(avg, wt, b2)


def kernel(x, emb_table, W, b):
    idx = x.reshape(-1).astype(jnp.int32)
    avg = _sc_gather_mean(idx, emb_table)
    wt = jnp.zeros((D, VPAD), jnp.float32).at[:, :V].set(W.T)
    b2 = jnp.zeros((1, VPAD), jnp.float32).at[:, :V].set(b[None, :])
    y = _tc_matmul(avg, wt, b2)
    return y[:, :V]


# speedup vs baseline: 2.5683x; 1.1214x over previous
import functools

import jax
import jax.numpy as jnp
from jax import lax
from jax.experimental import pallas as pl
from jax.experimental.pallas import tpu as pltpu
from jax.experimental.pallas import tpu_sc as plsc

B = 16384
L = 50
D = 64
V = 1000
VPAD = 1024

_info = plsc.get_sparse_core_info()
NC = _info.num_cores
NS = _info.num_subcores
NW = NC * NS
BPW = B // NW
CB = 8
NCHUNK = BPW // CB
IDX_PER_CHUNK = CB * L

_mesh = plsc.VectorSubcoreMesh(core_axis_name="c", subcore_axis_name="s")


NBUF = 2


@functools.partial(
    pl.kernel,
    mesh=_mesh,
    out_type=jax.ShapeDtypeStruct((B, D), jnp.float32),
    scratch_types=[
        pltpu.VMEM((BPW * L,), jnp.int32),
        [pltpu.VMEM((IDX_PER_CHUNK, D), jnp.float32) for _ in range(NBUF)],
        [pltpu.VMEM((CB, D), jnp.float32) for _ in range(NBUF)],
        [pltpu.SemaphoreType.DMA for _ in range(NBUF)],
        [pltpu.SemaphoreType.DMA for _ in range(NBUF)],
    ],
    compiler_params=pltpu.CompilerParams(use_tc_tiling_on_sc=False),
)
def _sc_gather_mean(idx_hbm, table_hbm, out_hbm, idx_v, rows_v, acc_v, gsem, osem):
    wid = lax.axis_index("s") * NC + lax.axis_index("c")
    base_row = wid * BPW
    pltpu.sync_copy(idx_hbm.at[pl.ds(base_row * L, BPW * L)], idx_v)

    def issue(c, b):
        pltpu.async_copy(
            table_hbm.at[idx_v.at[pl.ds(c * IDX_PER_CHUNK, IDX_PER_CHUNK)]],
            rows_v[b],
            gsem[b],
        )

    def wait_gather(b):
        pltpu.make_async_copy(
            table_hbm.at[idx_v.at[pl.ds(0, IDX_PER_CHUNK)]], rows_v[b], gsem[b]
        ).wait()

    def wait_out(b):
        pltpu.make_async_copy(acc_v[b], out_hbm.at[pl.ds(0, CB)], osem[b]).wait()

    def compute(b):
        rv, av = rows_v[b], acc_v[b]

        def rowloop(i, carry):
            accs = [rv[i * L, pl.ds(t * 16, 16)] for t in range(4)]
            for j in range(1, L):
                for t in range(4):
                    accs[t] = accs[t] + rv[i * L + j, pl.ds(t * 16, 16)]
            for t in range(4):
                av[i, pl.ds(t * 16, 16)] = accs[t] * (1.0 / L)
            return carry

        lax.fori_loop(0, CB, rowloop, 0)

    for b in range(NBUF):
        issue(b, b)

    def outer(g, carry):
        for b in range(NBUF):
            c = g * NBUF + b

            @pl.when(c >= NBUF)
            def _():
                wait_out(b)

            wait_gather(b)
            compute(b)

            @pl.when(c + NBUF < NCHUNK)
            def _():
                issue(c + NBUF, b)

            pltpu.async_copy(
                acc_v[b], out_hbm.at[pl.ds(base_row + c * CB, CB)], osem[b]
            )
        return carry

    lax.fori_loop(0, NCHUNK // NBUF, outer, 0)
    for b in range(NBUF):
        wait_out(b)


def _mm_body(avg_ref, wt_ref, b_ref, o_ref):
    o_ref[...] = (
        jnp.dot(avg_ref[...], wt_ref[...], preferred_element_type=jnp.float32)
        + b_ref[...]
    )


_BM = 2048


def _tc_matmul(avg, wt, b2):
    return pl.pallas_call(
        _mm_body,
        grid=(B // _BM,),
        in_specs=[
            pl.BlockSpec((_BM, D), lambda i: (i, 0)),
            pl.BlockSpec((D, VPAD), lambda i: (0, 0)),
            pl.BlockSpec((1, VPAD), lambda i: (0, 0)),
        ],
        out_specs=pl.BlockSpec((_BM, VPAD), lambda i: (i, 0)),
        out_shape=jax.ShapeDtypeStruct((B, VPAD), jnp.float32),
    )(avg, wt, b2)


def kernel(x, emb_table, W, b):
    idx = x.reshape(-1).astype(jnp.int32)
    avg = _sc_gather_mean(idx, emb_table)
    wt = jnp.zeros((D, VPAD), jnp.float32).at[:, :V].set(W.T)
    b2 = jnp.zeros((1, VPAD), jnp.float32).at[:, :V].set(b[None, :])
    y = _tc_matmul(avg, wt, b2)
    return y[:, :V]


# speedup vs baseline: 4.2825x; 1.6674x over previous
import functools

import jax
import jax.numpy as jnp
from jax import lax
from jax.experimental import pallas as pl
from jax.experimental.pallas import tpu as pltpu
from jax.experimental.pallas import tpu_sc as plsc

B = 16384
L = 50
D = 64
V = 1000
VPAD = 1024
NGRAM_VOCAB = 1000000
VB = 8192
NBLK = -(-NGRAM_VOCAB // VB)
LINES = NBLK * (VB // 2)

_info = plsc.get_sparse_core_info()
NC = _info.num_cores
NS = _info.num_subcores
NW = NC * NS
BPW = B // NW
CB = 8
NCHUNK = BPW // CB
IDX_PER_CHUNK = CB * L
NBUF = 2

_mesh = plsc.VectorSubcoreMesh(core_axis_name="c", subcore_axis_name="s")


def _repack_body(in_ref, out_ref):
    blk = in_ref[...]
    a = blk[:, 0 : VB // 2]
    b = blk[:, VB // 2 : VB]
    blk2 = jnp.concatenate([a, b], axis=0)
    eye = jnp.eye(2 * D, dtype=jnp.float32)
    dims = (((0,), (0,)), ((), ()))
    out_ref[...] = lax.dot_general(
        blk2, eye, dims, preferred_element_type=jnp.float32
    )


def _repack(table_t):
    return pl.pallas_call(
        _repack_body,
        grid=(NBLK,),
        in_specs=[pl.BlockSpec((D, VB), lambda i: (0, i))],
        out_specs=pl.BlockSpec((VB // 2, 2 * D), lambda i: (i, 0)),
        out_shape=jax.ShapeDtypeStruct((LINES, 2 * D), jnp.float32),
    )(table_t)


@functools.partial(
    pl.kernel,
    mesh=_mesh,
    out_type=jax.ShapeDtypeStruct((D, B), jnp.float32),
    scratch_types=[
        [pltpu.VMEM((IDX_PER_CHUNK,), jnp.int32) for _ in range(NBUF)],
        [pltpu.VMEM((IDX_PER_CHUNK, 2 * D), jnp.float32) for _ in range(NBUF)],
        pltpu.VMEM((D, 128), jnp.float32),
        [pltpu.VMEM((IDX_PER_CHUNK,), jnp.int32) for _ in range(NBUF)],
        [pltpu.VMEM((IDX_PER_CHUNK,), jnp.int32) for _ in range(NBUF)],
        [pltpu.SemaphoreType.DMA for _ in range(NBUF)],
    ],
    compiler_params=pltpu.CompilerParams(
        use_tc_tiling_on_sc=True, needs_layout_passes=False
    ),
)
def _sc_gather_mean(
    idx_hbm, table_hbm, out_hbm, idxc_v, rows_v, acc_v, line_v, lbase_v, gsem
):
    wid = lax.axis_index("s") * NC + lax.axis_index("c")
    base_row = wid * BPW
    _iota = lax.iota(jnp.int32, 16)
    GB = 128 // CB

    def prep(c, b):
        pltpu.sync_copy(
            idx_hbm.at[pl.ds((base_row + c * CB) * L, IDX_PER_CHUNK)], idxc_v[b]
        )

        def vecloop(m, carry):
            v = idxc_v[b][pl.ds(m * 16, 16)]
            blk = lax.shift_right_logical(v, 13)
            w = v & (VB // 2 - 1)
            half = lax.shift_right_logical(v, 12) & 1
            line_v[b][pl.ds(m * 16, 16)] = blk * (VB // 2) + w
            lbase_v[b][pl.ds(m * 16, 16)] = half * D
            return carry

        lax.fori_loop(0, IDX_PER_CHUNK // 16, vecloop, 0)

    def issue(b):
        pltpu.async_copy(table_hbm.at[line_v[b]], rows_v[b], gsem[b])

    def wait_gather(b):
        pltpu.make_async_copy(table_hbm.at[line_v[b]], rows_v[b], gsem[b]).wait()

    def compute(b, c):
        rv, av, lb = rows_v[b], acc_v, lbase_v[b]
        col0 = (c % GB) * CB

        def rowloop(i2, carry):
            accs = [[None] * 4 for _ in range(2)]
            lbv = [
                [lb[pl.ds((2 * i2 + u) * L + q * 16, 16)] for q in range(4)]
                for u in range(2)
            ]
            for j in range(L):
                jsplat = jnp.full((16,), j % 16, jnp.int32)
                for u in range(2):
                    r = (2 * i2 + u) * L + j
                    rsplat = jnp.full((16,), r, jnp.int32)
                    lbj = jnp.take(lbv[u][j // 16], jsplat)
                    for t in range(4):
                        lanes = lbj + (_iota + t * 16)
                        val = plsc.load_gather(rv, [rsplat, lanes])
                        accs[u][t] = val if j == 0 else accs[u][t] + val
            for u in range(2):
                for t in range(4):
                    plsc.store_scatter(
                        av,
                        [
                            _iota + t * 16,
                            jnp.full((16,), col0 + 2 * i2 + u, jnp.int32),
                        ],
                        accs[u][t] * (1.0 / L),
                    )
            return carry

        lax.fori_loop(0, CB // 2, rowloop, 0)

    for b in range(NBUF):
        prep(b, b)
        issue(b)

    def outer(g, carry):
        for b in range(NBUF):
            c = g * NBUF + b

            wait_gather(b)
            compute(b, c)

            @pl.when(c + NBUF < NCHUNK)
            def _():
                prep(c + NBUF, b)
                issue(b)

            @pl.when(c % GB == GB - 1)
            def _():
                start = pl.multiple_of(base_row + (c - (GB - 1)) * CB, 128)
                pltpu.sync_copy(acc_v, out_hbm.at[:, pl.ds(start, 128)])
        return carry

    lax.fori_loop(0, NCHUNK // NBUF, outer, 0)


def _mm_body(wt_ref, avgt_ref, b_ref, o_ref):
    dims = (((0,), (0,)), ((), ()))
    o_ref[...] = (
        lax.dot_general(
            wt_ref[...], avgt_ref[...], dims, preferred_element_type=jnp.float32
        )
        + b_ref[...][:, 0:1]
    )


_BM = 2048


def _tc_matmul(wt, avgt, b2):
    return pl.pallas_call(
        _mm_body,
        grid=(B // _BM,),
        in_specs=[
            pl.BlockSpec((D, V), lambda i: (0, 0)),
            pl.BlockSpec((D, _BM), lambda i: (0, i)),
            pl.BlockSpec((V, 128), lambda i: (0, 0)),
        ],
        out_specs=pl.BlockSpec((V, _BM), lambda i: (0, i)),
        out_shape=jax.ShapeDtypeStruct((V, B), jnp.float32),
    )(wt, avgt, b2)


def kernel(x, emb_table, W, b):
    idx = x.reshape(-1).astype(jnp.int32)
    table2 = _repack(emb_table.T)
    avgt = _sc_gather_mean(idx, table2)
    b2 = jnp.broadcast_to(b.reshape(V, 1), (V, 128))
    yt = _tc_matmul(W.T, avgt, b2)
    return yt.T
